# baseline (speedup 1.0000x reference)
import functools
import jax
import jax.numpy as jnp
from jax import lax
from jax.experimental import pallas as pl
from jax.experimental.pallas import tpu as pltpu
from jax.experimental.pallas import tpu_sc as plsc

N, T, P, D = 8, 48, 325, 512
TOK = N * T * P
NW = 32
TPW = TOK // NW
PPW = (N * T) // NW
TPADW = 3936
L = 16
CH = D // L
SUB = 8
GSLICE = 3 * D - (CH - 1) * L
TSIZES = (80, 80, 80, 80)
BUFROWS = 80

_mesh = plsc.VectorSubcoreMesh(core_axis_name="c", subcore_axis_name="s")


@functools.partial(
    pl.kernel,
    mesh=_mesh,
    compiler_params=pltpu.CompilerParams(needs_layout_passes=False),
    out_type=(jax.ShapeDtypeStruct((N, T, P, D), jnp.float32),
              jax.ShapeDtypeStruct((N, T, 8, D), jnp.float32)),
    scratch_types=[
        pltpu.VMEM((TPADW,), jnp.float32),
        pltpu.VMEM((TPADW,), jnp.float32),
        pltpu.VMEM((TPADW,), jnp.float32),
        pltpu.VMEM((TPADW,), jnp.float32),
        pltpu.VMEM((TPADW,), jnp.int32),
        pltpu.VMEM((D,), jnp.float32),
        pltpu.VMEM((D,), jnp.float32),
        pltpu.VMEM((3 * D,), jnp.float32),
        pltpu.VMEM((2 * BUFROWS, D), jnp.float32),
        pltpu.VMEM((L, D), jnp.float32),
        pltpu.SemaphoreType.DMA,
        pltpu.SemaphoreType.DMA,
        pltpu.SemaphoreType.DMA,
        pltpu.SemaphoreType.DMA,
        pltpu.SemaphoreType.DMA,
    ],
)
def _sc_embed(t_hbm, v_hbm, m_hbm, tw_hbm, vw_hbm, base_hbm,
              out_hbm, tail_hbm,
              t_v, v_v, m_v, b_v, s_v, tw_v, vw_v, base_v, obuf, tbuf,
              sem0, sem1, sem2, sem3, sem4):
    wid = lax.axis_index("s") * 2 + lax.axis_index("c")
    sems = (sem0, sem1, sem2, sem3)

    pltpu.sync_copy(t_hbm.at[pl.ds(wid * TPADW, TPADW)], t_v)
    pltpu.sync_copy(v_hbm.at[pl.ds(wid * TPADW, TPADW)], v_v)
    pltpu.sync_copy(m_hbm.at[pl.ds(wid * TPADW, TPADW)], m_v)
    pltpu.sync_copy(tw_hbm, tw_v)
    pltpu.sync_copy(vw_hbm, vw_v)
    pltpu.sync_copy(base_hbm, base_v)

    zf = jnp.zeros((L,), jnp.float32)
    s_emp = jnp.full((L,), D, jnp.int32)
    s_unm = jnp.full((L,), 2 * D, jnp.int32)
    s_val = jnp.zeros((L,), jnp.int32)
    iota16 = lax.iota(jnp.int32, L)

    def p1(j, carry):
        sl = pl.ds(j * L, L)
        v = v_v[sl]
        m = m_v[sl]
        inval = v != v
        notmon = m == zf
        b_v[sl] = jnp.where(inval | notmon, zf, v)
        s_v[sl] = jnp.where(notmon, s_unm, jnp.where(inval, s_emp, s_val))
        return carry

    lax.fori_loop(0, TPADW // L, p1, 0)

    def per_group_ref(lg, args, dst):
        toff0, brow = args
        toff = toff0 + lg * L
        tvec = t_v[pl.ds(toff, L)]
        bvec = b_v[pl.ds(toff, L)]
        svec = s_v[pl.ds(toff, L)]
        obase = brow + lg * L
        for sub in range(L // SUB):
            A = [jnp.full((L,), tvec[sub * SUB + j]) for j in range(SUB)]
            B = [jnp.full((L,), bvec[sub * SUB + j]) for j in range(SUB)]
            IX = [iota16 + svec[sub * SUB + j] for j in range(SUB)]
            rbase = obase + sub * SUB

            def per_chunk(k2, c2):
                ks = [2 * k2, 2 * k2 + 1]
                tws = [tw_v[pl.ds(k * L, L)] for k in ks]
                vws = [vw_v[pl.ds(k * L, L)] for k in ks]
                wins = [base_v.at[pl.ds(k * L, GSLICE)] for k in ks]
                bks = [plsc.load_gather(wins[h], [IX[j]])
                       for h in range(2) for j in range(SUB)]
                m1 = [A[j] * tws[h] for h in range(2) for j in range(SUB)]
                m2 = [B[j] * vws[h] for h in range(2) for j in range(SUB)]
                s1 = [a + b for a, b in zip(m1, m2)]
                s2 = [a + b for a, b in zip(s1, bks)]
                for h in range(2):
                    for j in range(SUB):
                        dst[rbase + j, pl.ds(ks[h] * L, L)] = s2[h * SUB + j]
                return c2

            lax.fori_loop(0, CH // 2, per_chunk, 0, unroll=2)
        return args

    def per_group(lg, args):
        return per_group_ref(lg, args, obuf)

    def per_group_tail(lg, args):
        return per_group_ref(lg, args, tbuf)

    def do_plate(i, carry):
        plate = wid * PPW + i
        n = plate // T
        t = plate % T
        tbase = i * P
        for ti in range(4):
            rows = TSIZES[ti]
            brow = (ti % 2) * BUFROWS
            if ti >= 2:
                pltpu.make_async_copy(
                    obuf.at[pl.ds(0, TSIZES[ti - 2]), :],
                    out_hbm.at[n, t, pl.ds(0, TSIZES[ti - 2]), :],
                    sems[ti - 2]).wait()
            else:
                @pl.when(i > 0)
                def _():
                    pltpu.make_async_copy(
                        obuf.at[pl.ds(0, TSIZES[ti + 2]), :],
                        out_hbm.at[n, t, pl.ds(0, TSIZES[ti + 2]), :],
                        sems[ti + 2]).wait()

            lax.fori_loop(0, 5, per_group, (tbase + ti * 80, brow))
            pltpu.make_async_copy(
                obuf.at[pl.ds(brow, rows), :],
                out_hbm.at[n, t, pl.ds(ti * 80, rows), :],
                sems[ti]).start()

        @pl.when(i > 0)
        def _():
            pltpu.make_async_copy(tbuf.at[pl.ds(0, 8), :],
                                  tail_hbm.at[n, t], sem4).wait()

        lax.fori_loop(0, 1, per_group_tail, (tbase + 320, 0))
        pltpu.make_async_copy(tbuf.at[pl.ds(0, 8), :],
                              tail_hbm.at[n, t], sem4).start()
        return carry

    lax.fori_loop(0, PPW, do_plate, 0)

    lastn = (wid * PPW + PPW - 1) // T
    lastt = (wid * PPW + PPW - 1) % T
    for ti in (2, 3):
        pltpu.make_async_copy(
            obuf.at[pl.ds(0, TSIZES[ti]), :],
            out_hbm.at[lastn, lastt, pl.ds(0, TSIZES[ti]), :],
            sems[ti]).wait()
    pltpu.make_async_copy(tbuf.at[pl.ds(0, 8), :],
                          tail_hbm.at[lastn, lastt], sem4).wait()


def kernel(x, monitor_mask, time_emb_w, time_emb_b, value_emb_w, value_emb_b,
           empty_token, unmonitored_token):
    value = x[..., 0].reshape(NW, TPW)
    time = x[..., 1].reshape(NW, TPW)
    mon = monitor_mask.reshape(NW, TPW).astype(jnp.float32)
    pad = ((0, 0), (0, TPADW - TPW))
    tb = time_emb_b.reshape(D)
    base = jnp.concatenate([
        tb + value_emb_b.reshape(D),
        tb + empty_token,
        tb + unmonitored_token,
    ])
    out, tail = _sc_embed(jnp.pad(time, pad).reshape(-1),
                          jnp.pad(value, pad).reshape(-1),
                          jnp.pad(mon, pad).reshape(-1),
                          time_emb_w.reshape(D), value_emb_w.reshape(D), base)
    return _stitch_tail(tail, out)


def _stitch_body(tail_ref, main_ref, out_ref):
    out_ref[...] = tail_ref[...]


def _stitch_tail(tail, main):
    return pl.pallas_call(
        _stitch_body,
        grid=(N, T),
        in_specs=[pl.BlockSpec((1, 1, 8, D), lambda n, t: (n, t, 0, 0)),
                  pl.BlockSpec(memory_space=pl.MemorySpace.ANY)],
        out_specs=pl.BlockSpec((1, 1, 8, D), lambda n, t: (n, t, 40, 0)),
        out_shape=jax.ShapeDtypeStruct((N, T, P, D), jnp.float32),
        input_output_aliases={1: 0},
    )(tail, main)


# speedup vs baseline: 1.4053x; 1.4053x over previous
import functools
import jax
import jax.numpy as jnp
from jax import lax
from jax.experimental import pallas as pl
from jax.experimental.pallas import tpu as pltpu
from jax.experimental.pallas import tpu_sc as plsc

N, T, P, D = 8, 48, 325, 512
TOK = N * T * P
NW = 32
TPW = TOK // NW
PPW = (N * T) // NW
TPADW = 3936
L = 16
CH = D // L
SUB = 8
GSLICE = 3 * D - (CH - 1) * L
TSIZES = (80, 80, 80, 80)
BUFROWS = 80

_mesh = plsc.VectorSubcoreMesh(core_axis_name="c", subcore_axis_name="s")


@functools.partial(
    pl.kernel,
    mesh=_mesh,
    compiler_params=pltpu.CompilerParams(needs_layout_passes=False),
    out_type=jax.ShapeDtypeStruct((N, T, 328, D), jnp.float32),
    scratch_types=[
        pltpu.VMEM((TPADW,), jnp.float32),
        pltpu.VMEM((TPADW,), jnp.float32),
        pltpu.VMEM((TPADW,), jnp.float32),
        pltpu.VMEM((TPADW,), jnp.float32),
        pltpu.VMEM((TPADW,), jnp.int32),
        pltpu.VMEM((D,), jnp.float32),
        pltpu.VMEM((D,), jnp.float32),
        pltpu.VMEM((3 * D,), jnp.float32),
        pltpu.VMEM((2 * BUFROWS, D), jnp.float32),
        pltpu.VMEM((L, D), jnp.float32),
        pltpu.SemaphoreType.DMA,
        pltpu.SemaphoreType.DMA,
        pltpu.SemaphoreType.DMA,
        pltpu.SemaphoreType.DMA,
        pltpu.SemaphoreType.DMA,
    ],
)
def _sc_embed(t_hbm, v_hbm, m_hbm, tw_hbm, vw_hbm, base_hbm,
              out_hbm,
              t_v, v_v, m_v, b_v, s_v, tw_v, vw_v, base_v, obuf, tbuf,
              sem0, sem1, sem2, sem3, sem4):
    wid = lax.axis_index("s") * 2 + lax.axis_index("c")
    sems = (sem0, sem1, sem2, sem3)

    pltpu.sync_copy(t_hbm.at[pl.ds(wid * TPADW, TPADW)], t_v)
    pltpu.sync_copy(v_hbm.at[pl.ds(wid * TPADW, TPADW)], v_v)
    pltpu.sync_copy(m_hbm.at[pl.ds(wid * TPADW, TPADW)], m_v)
    pltpu.sync_copy(tw_hbm, tw_v)
    pltpu.sync_copy(vw_hbm, vw_v)
    pltpu.sync_copy(base_hbm, base_v)

    zf = jnp.zeros((L,), jnp.float32)
    s_emp = jnp.full((L,), D, jnp.int32)
    s_unm = jnp.full((L,), 2 * D, jnp.int32)
    s_val = jnp.zeros((L,), jnp.int32)
    iota16 = lax.iota(jnp.int32, L)

    def p1(j, carry):
        sl = pl.ds(j * L, L)
        v = v_v[sl]
        m = m_v[sl]
        inval = v != v
        notmon = m == zf
        b_v[sl] = jnp.where(inval | notmon, zf, v)
        s_v[sl] = jnp.where(notmon, s_unm, jnp.where(inval, s_emp, s_val))
        return carry

    lax.fori_loop(0, TPADW // L, p1, 0)

    def per_group_ref(lg, args, dst):
        toff0, brow = args
        toff = toff0 + lg * L
        tvec = t_v[pl.ds(toff, L)]
        bvec = b_v[pl.ds(toff, L)]
        svec = s_v[pl.ds(toff, L)]
        obase = brow + lg * L
        for sub in range(L // SUB):
            A = [jnp.full((L,), tvec[sub * SUB + j]) for j in range(SUB)]
            B = [jnp.full((L,), bvec[sub * SUB + j]) for j in range(SUB)]
            IX = [iota16 + svec[sub * SUB + j] for j in range(SUB)]
            rbase = obase + sub * SUB

            def per_chunk(k2, c2):
                ks = [2 * k2, 2 * k2 + 1]
                tws = [tw_v[pl.ds(k * L, L)] for k in ks]
                vws = [vw_v[pl.ds(k * L, L)] for k in ks]
                wins = [base_v.at[pl.ds(k * L, GSLICE)] for k in ks]
                bks = [plsc.load_gather(wins[h], [IX[j]])
                       for h in range(2) for j in range(SUB)]
                m1 = [A[j] * tws[h] for h in range(2) for j in range(SUB)]
                m2 = [B[j] * vws[h] for h in range(2) for j in range(SUB)]
                s1 = [a + b for a, b in zip(m1, m2)]
                s2 = [a + b for a, b in zip(s1, bks)]
                for h in range(2):
                    for j in range(SUB):
                        dst[rbase + j, pl.ds(ks[h] * L, L)] = s2[h * SUB + j]
                return c2

            lax.fori_loop(0, CH // 2, per_chunk, 0, unroll=2)
        return args

    def per_group(lg, args):
        return per_group_ref(lg, args, obuf)

    def per_group_tail(lg, args):
        return per_group_ref(lg, args, tbuf)

    def do_plate(i, carry):
        plate = wid * PPW + i
        n = plate // T
        t = plate % T
        tbase = i * P
        for ti in range(4):
            rows = TSIZES[ti]
            brow = (ti % 2) * BUFROWS
            if ti >= 2:
                pltpu.make_async_copy(
                    obuf.at[pl.ds(0, TSIZES[ti - 2]), :],
                    out_hbm.at[n, t, pl.ds(0, TSIZES[ti - 2]), :],
                    sems[ti - 2]).wait()
            else:
                @pl.when(i > 0)
                def _():
                    pltpu.make_async_copy(
                        obuf.at[pl.ds(0, TSIZES[ti + 2]), :],
                        out_hbm.at[n, t, pl.ds(0, TSIZES[ti + 2]), :],
                        sems[ti + 2]).wait()

            lax.fori_loop(0, 5, per_group, (tbase + ti * 80, brow))
            pltpu.make_async_copy(
                obuf.at[pl.ds(brow, rows), :],
                out_hbm.at[n, t, pl.ds(ti * 80, rows), :],
                sems[ti]).start()

        @pl.when(i > 0)
        def _():
            pltpu.make_async_copy(tbuf.at[pl.ds(0, 8), :],
                                  out_hbm.at[n, t, pl.ds(320, 8), :],
                                  sem4).wait()

        lax.fori_loop(0, 1, per_group_tail, (tbase + 320, 0))
        pltpu.make_async_copy(tbuf.at[pl.ds(0, 8), :],
                              out_hbm.at[n, t, pl.ds(320, 8), :],
                              sem4).start()
        return carry

    lax.fori_loop(0, PPW, do_plate, 0)

    lastn = (wid * PPW + PPW - 1) // T
    lastt = (wid * PPW + PPW - 1) % T
    for ti in (2, 3):
        pltpu.make_async_copy(
            obuf.at[pl.ds(0, TSIZES[ti]), :],
            out_hbm.at[lastn, lastt, pl.ds(0, TSIZES[ti]), :],
            sems[ti]).wait()
    pltpu.make_async_copy(tbuf.at[pl.ds(0, 8), :],
                          out_hbm.at[lastn, lastt, pl.ds(320, 8), :],
                          sem4).wait()


def kernel(x, monitor_mask, time_emb_w, time_emb_b, value_emb_w, value_emb_b,
           empty_token, unmonitored_token):
    value = x[..., 0].reshape(NW, TPW)
    time = x[..., 1].reshape(NW, TPW)
    mon = monitor_mask.reshape(NW, TPW).astype(jnp.float32)
    pad = ((0, 0), (0, TPADW - TPW))
    tb = time_emb_b.reshape(D)
    base = jnp.concatenate([
        tb + value_emb_b.reshape(D),
        tb + empty_token,
        tb + unmonitored_token,
    ])
    out = _sc_embed(jnp.pad(time, pad).reshape(-1),
                    jnp.pad(value, pad).reshape(-1),
                    jnp.pad(mon, pad).reshape(-1),
                    time_emb_w.reshape(D), value_emb_w.reshape(D), base)
    return out[:, :, :P, :]
